# Initial kernel scaffold; baseline (speedup 1.0000x reference)
import jax
import jax.numpy as jnp
from jax.experimental import pallas as pl


def kernel(cls_scores, bbox_preds):
    raise NotImplementedError("write your pallas kernel here")


# speedup vs baseline: 3.1216x; 3.1216x over previous
import numpy as np
import jax
import jax.numpy as jnp
from jax.experimental import pallas as pl

_FEAT_STRIDE = 16
_IMAGE_SIZE = 1920
_NMS_POST = 2000
_THRESH = 0.6
_MAP_H = 120
_MAP_W = 120
_NPAD = 2048


def _base_anchors():
    base_size = 16
    ratios = np.array([0.5, 1.0, 2.0])
    scales = np.array([8.0, 16.0, 32.0])
    ctr = 0.5 * (base_size - 1)

    def mk(ws, hs, x_ctr, y_ctr):
        ws = np.asarray(ws, dtype=np.float64)[:, None]
        hs = np.asarray(hs, dtype=np.float64)[:, None]
        return np.hstack((x_ctr - 0.5 * (ws - 1), y_ctr - 0.5 * (hs - 1),
                          x_ctr + 0.5 * (ws - 1), y_ctr + 0.5 * (hs - 1)))

    size = float(base_size) * float(base_size)
    ws = np.round(np.sqrt(size / ratios))
    hs = np.round(ws * ratios)
    ratio_anchors = mk(ws, hs, ctr, ctr)
    out = []
    for ra in ratio_anchors:
        aw = ra[2] - ra[0] + 1.0
        ah = ra[3] - ra[1] + 1.0
        axc = ra[0] + 0.5 * (aw - 1)
        ayc = ra[1] + 0.5 * (ah - 1)
        out.append(mk(aw * scales, ah * scales, axc, ayc))
    return np.vstack(out).astype(np.float32)


def _grid_anchors():
    anchors = _base_anchors()
    shift_x = (np.arange(_MAP_W) * _FEAT_STRIDE).astype(np.float32)
    shift_y = (np.arange(_MAP_H) * _FEAT_STRIDE).astype(np.float32)
    sx, sy = np.meshgrid(shift_x, shift_y)
    shifts = np.stack((sx.ravel(), sy.ravel(), sx.ravel(), sy.ravel()), axis=1)
    return (anchors[None, :, :] + shifts[:, None, :]).reshape(-1, 4)


_ANCHORS = _grid_anchors()


def _nms_body(feats_ref, out_ref):
    dx = feats_ref[0]
    dy = feats_ref[1]
    dw = feats_ref[2]
    dh = feats_ref[3]
    ax1 = feats_ref[4]
    ay1 = feats_ref[5]
    ax2 = feats_ref[6]
    ay2 = feats_ref[7]

    w = ax2 - ax1 + 1.0
    h = ay2 - ay1 + 1.0
    cx = ax1 + 0.5 * w
    cy = ay1 + 0.5 * h
    pcx = dx * w + cx
    pcy = dy * h + cy
    pw = jnp.exp(dw) * w
    ph = jnp.exp(dh) * h
    hi = float(_IMAGE_SIZE - 1)
    x1 = jnp.clip(pcx - 0.5 * pw, 0.0, hi)
    y1 = jnp.clip(pcy - 0.5 * ph, 0.0, hi)
    x2 = jnp.clip(pcx + 0.5 * pw, 0.0, hi)
    y2 = jnp.clip(pcy + 0.5 * ph, 0.0, hi)
    areas = (x2 - x1 + 1.0) * (y2 - y1 + 1.0)

    idx = (jax.lax.broadcasted_iota(jnp.int32, (16, 128), 0) * 128
           + jax.lax.broadcasted_iota(jnp.int32, (16, 128), 1))
    lane = jax.lax.broadcasted_iota(jnp.int32, (1, 128), 1)
    keep0 = (idx < _NMS_POST).astype(jnp.float32)

    out_ref[...] = jnp.zeros((_NPAD, 128), jnp.float32)

    def body(i, carry):
        cnt, keep = carry
        oh = (idx == i).astype(jnp.float32)
        xi1 = jnp.sum(x1 * oh)
        yi1 = jnp.sum(y1 * oh)
        xi2 = jnp.sum(x2 * oh)
        yi2 = jnp.sum(y2 * oh)
        ai = jnp.sum(areas * oh)
        ki = jnp.sum(keep * oh)
        kept = ki > 0.5
        iw = jnp.maximum(jnp.minimum(x2, xi2) - jnp.maximum(x1, xi1) + 1.0, 0.0)
        ih = jnp.maximum(jnp.minimum(y2, yi2) - jnp.maximum(y1, yi1) + 1.0, 0.0)
        inter = iw * ih
        iou = inter / (ai + areas - inter)
        sup = (iou > _THRESH) & (idx > i) & kept
        keep = jnp.where(sup, 0.0, keep)
        row = jnp.where(lane == 0, xi1,
              jnp.where(lane == 1, yi1,
              jnp.where(lane == 2, xi2,
              jnp.where(lane == 3, yi2, 0.0)))) * ki
        out_ref[pl.ds(cnt, 1), :] = row
        return cnt + kept.astype(jnp.int32), keep

    jax.lax.fori_loop(0, _NMS_POST, body, (jnp.int32(0), keep0))


def kernel(cls_scores, bbox_preds):
    batch = cls_scores.shape[0]
    scores = jnp.transpose(cls_scores[..., 0], (0, 2, 3, 1)).reshape(batch, -1)
    deltas = jnp.transpose(bbox_preds, (0, 2, 3, 1, 4)).reshape(batch, -1, 4)

    _, order = jax.lax.top_k(scores, _NMS_POST)
    d_sel = jnp.take_along_axis(deltas, order[..., None], axis=1)
    a_sel = jnp.asarray(_ANCHORS)[order]

    feats = jnp.concatenate(
        (jnp.moveaxis(d_sel, -1, 1), jnp.moveaxis(a_sel, -1, 1)), axis=1
    )
    feats = jnp.pad(feats, ((0, 0), (0, 0), (0, _NPAD - _NMS_POST)))
    feats = feats.reshape(batch, 8, 16, 128)

    out = pl.pallas_call(
        _nms_body,
        grid=(batch,),
        in_specs=[pl.BlockSpec((None, 8, 16, 128), lambda b: (b, 0, 0, 0))],
        out_specs=pl.BlockSpec((None, _NPAD, 128), lambda b: (b, 0, 0)),
        out_shape=jax.ShapeDtypeStruct((batch, _NPAD, 128), jnp.float32),
    )(feats)

    boxes = out[:, :_NMS_POST, :4]
    bidx = jnp.broadcast_to(
        jnp.arange(batch, dtype=jnp.float32)[:, None, None],
        (batch, _NMS_POST, 1))
    return jnp.concatenate((bidx, boxes), axis=-1)


# speedup vs baseline: 3.1945x; 1.0234x over previous
import numpy as np
import jax
import jax.numpy as jnp
from jax.experimental import pallas as pl
from jax.experimental.pallas import tpu as pltpu

_FEAT_STRIDE = 16
_IMAGE_SIZE = 1920
_NMS_POST = 2000
_THRESH = 0.6
_MAP_H = 120
_MAP_W = 120
_NPAD = 2048


def _base_anchors():
    base_size = 16
    ratios = np.array([0.5, 1.0, 2.0])
    scales = np.array([8.0, 16.0, 32.0])
    ctr = 0.5 * (base_size - 1)

    def mk(ws, hs, x_ctr, y_ctr):
        ws = np.asarray(ws, dtype=np.float64)[:, None]
        hs = np.asarray(hs, dtype=np.float64)[:, None]
        return np.hstack((x_ctr - 0.5 * (ws - 1), y_ctr - 0.5 * (hs - 1),
                          x_ctr + 0.5 * (ws - 1), y_ctr + 0.5 * (hs - 1)))

    size = float(base_size) * float(base_size)
    ws = np.round(np.sqrt(size / ratios))
    hs = np.round(ws * ratios)
    ratio_anchors = mk(ws, hs, ctr, ctr)
    out = []
    for ra in ratio_anchors:
        aw = ra[2] - ra[0] + 1.0
        ah = ra[3] - ra[1] + 1.0
        axc = ra[0] + 0.5 * (aw - 1)
        ayc = ra[1] + 0.5 * (ah - 1)
        out.append(mk(aw * scales, ah * scales, axc, ayc))
    return np.vstack(out).astype(np.float32)


def _grid_anchors():
    anchors = _base_anchors()
    shift_x = (np.arange(_MAP_W) * _FEAT_STRIDE).astype(np.float32)
    shift_y = (np.arange(_MAP_H) * _FEAT_STRIDE).astype(np.float32)
    sx, sy = np.meshgrid(shift_x, shift_y)
    shifts = np.stack((sx.ravel(), sy.ravel(), sx.ravel(), sy.ravel()), axis=1)
    return (anchors[None, :, :] + shifts[:, None, :]).reshape(-1, 4)


_ANCHORS = _grid_anchors()


def _decode_body(feats_ref, out_ref):
    dx = feats_ref[:, 0]
    dy = feats_ref[:, 1]
    dw = feats_ref[:, 2]
    dh = feats_ref[:, 3]
    ax1 = feats_ref[:, 4]
    ay1 = feats_ref[:, 5]
    ax2 = feats_ref[:, 6]
    ay2 = feats_ref[:, 7]

    w = ax2 - ax1 + 1.0
    h = ay2 - ay1 + 1.0
    cx = ax1 + 0.5 * w
    cy = ay1 + 0.5 * h
    pcx = dx * w + cx
    pcy = dy * h + cy
    pw = jnp.exp(dw) * w
    ph = jnp.exp(dh) * h
    hi = float(_IMAGE_SIZE - 1)
    x1 = jnp.clip(pcx - 0.5 * pw, 0.0, hi)
    y1 = jnp.clip(pcy - 0.5 * ph, 0.0, hi)
    x2 = jnp.clip(pcx + 0.5 * pw, 0.0, hi)
    y2 = jnp.clip(pcy + 0.5 * ph, 0.0, hi)
    out_ref[:, 0] = x1
    out_ref[:, 1] = y1
    out_ref[:, 2] = x2
    out_ref[:, 3] = y2
    out_ref[:, 4] = (x2 - x1 + 1.0) * (y2 - y1 + 1.0)


def _nms_body(vec_ref, col_ref, out_ref):
    x1 = vec_ref[0]
    y1 = vec_ref[1]
    x2 = vec_ref[2]
    y2 = vec_ref[3]
    areas = vec_ref[4]

    idx = (jax.lax.broadcasted_iota(jnp.int32, (16, 128), 0) * 128
           + jax.lax.broadcasted_iota(jnp.int32, (16, 128), 1))
    lane = jax.lax.broadcasted_iota(jnp.int32, (1, 128), 1)
    keep0 = (idx < _NMS_POST).astype(jnp.float32)

    out_ref[...] = jnp.zeros((_NPAD, 128), jnp.float32)

    def body(i, carry):
        cnt, keep = carry
        xi1 = col_ref[0, i]
        yi1 = col_ref[1, i]
        xi2 = col_ref[2, i]
        yi2 = col_ref[3, i]
        ai = col_ref[4, i]
        ki = jnp.sum(keep * (idx == i).astype(jnp.float32))
        kept = ki > 0.5
        iw = jnp.maximum(jnp.minimum(x2, xi2) - jnp.maximum(x1, xi1) + 1.0, 0.0)
        ih = jnp.maximum(jnp.minimum(y2, yi2) - jnp.maximum(y1, yi1) + 1.0, 0.0)
        inter = iw * ih
        iou = inter / (ai + areas - inter)
        sup = (iou > _THRESH) & (idx > i) & kept
        keep = jnp.where(sup, 0.0, keep)
        row = jnp.where(lane == 0, xi1,
              jnp.where(lane == 1, yi1,
              jnp.where(lane == 2, xi2,
              jnp.where(lane == 3, yi2, 0.0)))) * ki
        out_ref[pl.ds(cnt, 1), :] = row
        return cnt + kept.astype(jnp.int32), keep

    jax.lax.fori_loop(0, _NMS_POST, body, (jnp.int32(0), keep0))


def kernel(cls_scores, bbox_preds):
    batch = cls_scores.shape[0]
    scores = jnp.transpose(cls_scores[..., 0], (0, 2, 3, 1)).reshape(batch, -1)
    deltas = jnp.transpose(bbox_preds, (0, 2, 3, 1, 4)).reshape(batch, -1, 4)

    _, order = jax.lax.top_k(scores, _NMS_POST)
    d_sel = jnp.take_along_axis(deltas, order[..., None], axis=1)
    a_sel = jnp.asarray(_ANCHORS)[order]

    feats = jnp.concatenate(
        (jnp.moveaxis(d_sel, -1, 1), jnp.moveaxis(a_sel, -1, 1)), axis=1
    )
    feats = jnp.pad(feats, ((0, 0), (0, 0), (0, _NPAD - _NMS_POST)))
    feats = feats.reshape(batch, 8, 16, 128)

    coords = pl.pallas_call(
        _decode_body,
        out_shape=jax.ShapeDtypeStruct((batch, 5, 16, 128), jnp.float32),
    )(feats)
    coords_flat = coords.reshape(batch, 5, _NPAD)

    out = pl.pallas_call(
        _nms_body,
        grid=(batch,),
        in_specs=[
            pl.BlockSpec((None, 5, 16, 128), lambda b: (b, 0, 0, 0)),
            pl.BlockSpec((None, 5, _NPAD), lambda b: (b, 0, 0),
                         memory_space=pltpu.SMEM),
        ],
        out_specs=pl.BlockSpec((None, _NPAD, 128), lambda b: (b, 0, 0)),
        out_shape=jax.ShapeDtypeStruct((batch, _NPAD, 128), jnp.float32),
    )(coords, coords_flat)

    boxes = out[:, :_NMS_POST, :4]
    bidx = jnp.broadcast_to(
        jnp.arange(batch, dtype=jnp.float32)[:, None, None],
        (batch, _NMS_POST, 1))
    return jnp.concatenate((bidx, boxes), axis=-1)


# speedup vs baseline: 3.3862x; 1.0600x over previous
import numpy as np
import jax
import jax.numpy as jnp
from jax.experimental import pallas as pl
from jax.experimental.pallas import tpu as pltpu

_FEAT_STRIDE = 16
_IMAGE_SIZE = 1920
_NMS_POST = 2000
_THRESH = 0.6
_MAP_H = 120
_MAP_W = 120
_NPAD = 2048


def _base_anchors():
    base_size = 16
    ratios = np.array([0.5, 1.0, 2.0])
    scales = np.array([8.0, 16.0, 32.0])
    ctr = 0.5 * (base_size - 1)

    def mk(ws, hs, x_ctr, y_ctr):
        ws = np.asarray(ws, dtype=np.float64)[:, None]
        hs = np.asarray(hs, dtype=np.float64)[:, None]
        return np.hstack((x_ctr - 0.5 * (ws - 1), y_ctr - 0.5 * (hs - 1),
                          x_ctr + 0.5 * (ws - 1), y_ctr + 0.5 * (hs - 1)))

    size = float(base_size) * float(base_size)
    ws = np.round(np.sqrt(size / ratios))
    hs = np.round(ws * ratios)
    ratio_anchors = mk(ws, hs, ctr, ctr)
    out = []
    for ra in ratio_anchors:
        aw = ra[2] - ra[0] + 1.0
        ah = ra[3] - ra[1] + 1.0
        axc = ra[0] + 0.5 * (aw - 1)
        ayc = ra[1] + 0.5 * (ah - 1)
        out.append(mk(aw * scales, ah * scales, axc, ayc))
    return np.vstack(out).astype(np.float32)


def _grid_anchors():
    anchors = _base_anchors()
    shift_x = (np.arange(_MAP_W) * _FEAT_STRIDE).astype(np.float32)
    shift_y = (np.arange(_MAP_H) * _FEAT_STRIDE).astype(np.float32)
    sx, sy = np.meshgrid(shift_x, shift_y)
    shifts = np.stack((sx.ravel(), sy.ravel(), sx.ravel(), sy.ravel()), axis=1)
    return (anchors[None, :, :] + shifts[:, None, :]).reshape(-1, 4)


_ANCHORS = _grid_anchors()


def _decode_body(feats_ref, out_ref):
    dx = feats_ref[:, 0]
    dy = feats_ref[:, 1]
    dw = feats_ref[:, 2]
    dh = feats_ref[:, 3]
    ax1 = feats_ref[:, 4]
    ay1 = feats_ref[:, 5]
    ax2 = feats_ref[:, 6]
    ay2 = feats_ref[:, 7]

    w = ax2 - ax1 + 1.0
    h = ay2 - ay1 + 1.0
    cx = ax1 + 0.5 * w
    cy = ay1 + 0.5 * h
    pcx = dx * w + cx
    pcy = dy * h + cy
    pw = jnp.exp(dw) * w
    ph = jnp.exp(dh) * h
    hi = float(_IMAGE_SIZE - 1)
    x1 = jnp.clip(pcx - 0.5 * pw, 0.0, hi)
    y1 = jnp.clip(pcy - 0.5 * ph, 0.0, hi)
    x2 = jnp.clip(pcx + 0.5 * pw, 0.0, hi)
    y2 = jnp.clip(pcy + 0.5 * ph, 0.0, hi)
    out_ref[:, 0] = x1
    out_ref[:, 1] = y1
    out_ref[:, 2] = x2
    out_ref[:, 3] = y2
    out_ref[:, 4] = (x2 - x1 + 1.0) * (y2 - y1 + 1.0)


def _nms_body(vec_ref, col_ref, out_ref):
    x1 = vec_ref[0]
    y1 = vec_ref[1]
    x2 = vec_ref[2]
    y2 = vec_ref[3]
    areas = vec_ref[4]

    idx = (jax.lax.broadcasted_iota(jnp.int32, (16, 128), 0) * 128
           + jax.lax.broadcasted_iota(jnp.int32, (16, 128), 1))
    lane = jax.lax.broadcasted_iota(jnp.int32, (1, 128), 1)
    keep0 = (idx < _NMS_POST).astype(jnp.float32)

    out_ref[...] = jnp.zeros((_NPAD, 128), jnp.float32)

    def body(i, carry):
        cnt, keep = carry
        xi1 = col_ref[0, i]
        yi1 = col_ref[1, i]
        xi2 = col_ref[2, i]
        yi2 = col_ref[3, i]
        ai = col_ref[4, i]
        ki = jnp.sum(keep * (idx == i).astype(jnp.float32))
        kept = ki > 0.5
        iw = jnp.maximum(jnp.minimum(x2, xi2) - jnp.maximum(x1, xi1) + 1.0, 0.0)
        ih = jnp.maximum(jnp.minimum(y2, yi2) - jnp.maximum(y1, yi1) + 1.0, 0.0)
        inter = iw * ih
        iou = inter / (ai + areas - inter)
        sup = (iou > _THRESH) & (idx > i) & kept
        keep = jnp.where(sup, 0.0, keep)
        row = jnp.where(lane == 0, xi1,
              jnp.where(lane == 1, yi1,
              jnp.where(lane == 2, xi2,
              jnp.where(lane == 3, yi2, 0.0)))) * ki
        out_ref[pl.ds(cnt, 1), :] = row
        return cnt + kept.astype(jnp.int32), keep

    jax.lax.fori_loop(0, _NMS_POST, body, (jnp.int32(0), keep0))


def kernel(cls_scores, bbox_preds):
    batch = cls_scores.shape[0]
    hw = _MAP_H * _MAP_W
    scores = jnp.transpose(cls_scores[..., 0], (0, 2, 3, 1)).reshape(batch, -1)
    _, order_m = jax.lax.top_k(scores, _NMS_POST)
    order_j = (order_m % 9) * hw + order_m // 9

    deltas = bbox_preds.reshape(batch, -1, 4)
    d_sel = jnp.take_along_axis(deltas, order_j[..., None], axis=1)
    a_sel = jnp.asarray(_ANCHORS)[order_m]

    feats = jnp.concatenate(
        (jnp.moveaxis(d_sel, -1, 1), jnp.moveaxis(a_sel, -1, 1)), axis=1
    )
    feats = jnp.pad(feats, ((0, 0), (0, 0), (0, _NPAD - _NMS_POST)))
    feats = feats.reshape(batch, 8, 16, 128)

    coords = pl.pallas_call(
        _decode_body,
        out_shape=jax.ShapeDtypeStruct((batch, 5, 16, 128), jnp.float32),
    )(feats)
    coords_flat = coords.reshape(batch, 5, _NPAD)

    out = pl.pallas_call(
        _nms_body,
        grid=(batch,),
        in_specs=[
            pl.BlockSpec((None, 5, 16, 128), lambda b: (b, 0, 0, 0)),
            pl.BlockSpec((None, 5, _NPAD), lambda b: (b, 0, 0),
                         memory_space=pltpu.SMEM),
        ],
        out_specs=pl.BlockSpec((None, _NPAD, 128), lambda b: (b, 0, 0)),
        out_shape=jax.ShapeDtypeStruct((batch, _NPAD, 128), jnp.float32),
    )(coords, coords_flat)

    boxes = out[:, :_NMS_POST, :4]
    bidx = jnp.broadcast_to(
        jnp.arange(batch, dtype=jnp.float32)[:, None, None],
        (batch, _NMS_POST, 1))
    return jnp.concatenate((bidx, boxes), axis=-1)
